# Initial kernel scaffold; baseline (speedup 1.0000x reference)
import jax
import jax.numpy as jnp
from jax.experimental import pallas as pl


def kernel(user, game, user_table, game_table, fc_w, fc_b):
    raise NotImplementedError("write your pallas kernel here")


# speedup vs baseline: 5.7920x; 5.7920x over previous
import functools

import jax
import jax.numpy as jnp
from jax import lax
from jax.experimental import pallas as pl
from jax.experimental.pallas import tpu as pltpu
from jax.experimental.pallas import tpu_sc as plsc

NC, NS = 2, 16
NW = NC * NS
B = 16384
D = 128
BPW = B // NW
C = 5


def _gather_body(user_t, game_t, user_idx, game_idx, uout, gout,
                 idx_v, rows_v, sem):
    wid = lax.axis_index("s") * NC + lax.axis_index("c")
    base = wid * BPW
    pltpu.sync_copy(user_idx.at[pl.ds(base, BPW)], idx_v)
    pltpu.async_copy(user_t.at[idx_v], rows_v, sem).wait()
    pltpu.sync_copy(rows_v, uout.at[pl.ds(base, BPW)])
    pltpu.sync_copy(game_idx.at[pl.ds(base, BPW)], idx_v)
    pltpu.async_copy(game_t.at[idx_v], rows_v, sem).wait()
    pltpu.sync_copy(rows_v, gout.at[pl.ds(base, BPW)])


_sc_gather = pl.kernel(
    _gather_body,
    out_type=(jax.ShapeDtypeStruct((B, D), jnp.float32),
              jax.ShapeDtypeStruct((B, D), jnp.float32)),
    mesh=plsc.VectorSubcoreMesh(core_axis_name="c", subcore_axis_name="s"),
    scratch_types=[
        pltpu.VMEM((BPW,), jnp.int32),
        pltpu.VMEM((BPW, D), jnp.float32),
        pltpu.SemaphoreType.DMA,
    ],
)


def _matmul_body(u_ref, g_ref, w1_ref, w2_ref, b_ref, o_ref):
    acc = jnp.dot(u_ref[...], w1_ref[...], preferred_element_type=jnp.float32)
    acc += jnp.dot(g_ref[...], w2_ref[...], preferred_element_type=jnp.float32)
    o_ref[...] = acc + b_ref[...]


def _tc_matmul(uemb, gemb, w1, w2, bias):
    bm = 2048
    grid = (B // bm,)
    return pl.pallas_call(
        _matmul_body,
        grid=grid,
        in_specs=[
            pl.BlockSpec((bm, D), lambda i: (i, 0)),
            pl.BlockSpec((bm, D), lambda i: (i, 0)),
            pl.BlockSpec((D, C), lambda i: (0, 0)),
            pl.BlockSpec((D, C), lambda i: (0, 0)),
            pl.BlockSpec((1, C), lambda i: (0, 0)),
        ],
        out_specs=pl.BlockSpec((bm, C), lambda i: (i, 0)),
        out_shape=jax.ShapeDtypeStruct((B, C), jnp.float32),
    )(uemb, gemb, w1, w2, bias)


def kernel(user, game, user_table, game_table, fc_w, fc_b):
    uemb, gemb = _sc_gather(user_table, game_table, user, game)
    w1 = fc_w[:, :D].T
    w2 = fc_w[:, D:].T
    return _tc_matmul(uemb, gemb, w1, w2, fc_b.reshape(1, C))
